# Initial kernel scaffold; baseline (speedup 1.0000x reference)
import jax
import jax.numpy as jnp
from jax.experimental import pallas as pl


def kernel(x, edge_index, W1, b1, Wmu, bmu, Wlv, blv):
    raise NotImplementedError("write your pallas kernel here")


# speedup vs baseline: 14.4057x; 14.4057x over previous
import functools

import jax
import jax.numpy as jnp
from jax import lax
from jax.experimental import pallas as pl
from jax.experimental.pallas import tpu as pltpu
from jax.experimental.pallas import tpu_sc as plsc

NC = 2
NS = 16
CHUNK = 128
N_PAD = 10240
ROWS_PER_TILE = N_PAD // NS
DEG_W = 16

_mesh = plsc.VectorSubcoreMesh(core_axis_name="c", subcore_axis_name="s")


def _make_sc_scatter(chunks_per_w, d):
  gather = chunks_per_w is not None

  @functools.partial(
      pl.kernel,
      mesh=_mesh,
      out_type=jax.ShapeDtypeStruct((NC * N_PAD, d), jnp.float32),
      scratch_types=[
          pltpu.VMEM((chunks_per_w if gather else 1, CHUNK), jnp.int32),
          pltpu.VMEM((chunks_per_w if gather else 1, CHUNK), jnp.int32),
          pltpu.VMEM((CHUNK, d), jnp.float32),
          pltpu.VMEM_SHARED((N_PAD, d), jnp.float32),
      ],
  )
  def k(yt_hbm, src_hbm, dst_hbm, zeros_hbm, out_hbm, srcv, dstv, buf, acc):
    c = lax.axis_index("c")
    s = lax.axis_index("s")
    w = c * NS + s
    pltpu.sync_copy(src_hbm.at[w], srcv)
    pltpu.sync_copy(dst_hbm.at[w], dstv)
    row0 = s * ROWS_PER_TILE
    pltpu.sync_copy(
        zeros_hbm.at[pl.ds(row0, ROWS_PER_TILE)],
        acc.at[pl.ds(row0, ROWS_PER_TILE)],
    )
    plsc.subcore_barrier()

    def body(j, carry):
      pltpu.sync_copy(yt_hbm.at[srcv.at[j]], buf)
      pltpu.sync_copy(buf, acc.at[dstv.at[j]], add=True)
      return carry

    lax.fori_loop(0, chunks_per_w, body, 0)
    plsc.subcore_barrier()
    pltpu.sync_copy(
        acc.at[pl.ds(row0, ROWS_PER_TILE)],
        out_hbm.at[pl.ds(c * N_PAD + row0, ROWS_PER_TILE)],
    )

  return k


def _make_sc_deg(chunks_per_w):
  @functools.partial(
      pl.kernel,
      mesh=_mesh,
      out_type=jax.ShapeDtypeStruct((NC * N_PAD, DEG_W), jnp.float32),
      scratch_types=[
          pltpu.VMEM((chunks_per_w, CHUNK), jnp.int32),
          pltpu.VMEM((CHUNK, DEG_W), jnp.float32),
          pltpu.VMEM_SHARED((N_PAD, DEG_W), jnp.float32),
      ],
  )
  def k(ones_hbm, dst_hbm, zeros_hbm, out_hbm, dstv, ones_v, acc):
    c = lax.axis_index("c")
    s = lax.axis_index("s")
    w = c * NS + s
    pltpu.sync_copy(dst_hbm.at[w], dstv)
    pltpu.sync_copy(ones_hbm, ones_v)
    row0 = s * ROWS_PER_TILE
    pltpu.sync_copy(
        zeros_hbm.at[pl.ds(row0, ROWS_PER_TILE)],
        acc.at[pl.ds(row0, ROWS_PER_TILE)],
    )
    plsc.subcore_barrier()

    def body(j, carry):
      pltpu.sync_copy(ones_v, acc.at[dstv.at[j]], add=True)
      return carry

    lax.fori_loop(0, chunks_per_w, body, 0)
    plsc.subcore_barrier()
    pltpu.sync_copy(
        acc.at[pl.ds(row0, ROWS_PER_TILE)],
        out_hbm.at[pl.ds(c * N_PAD + row0, ROWS_PER_TILE)],
    )

  return k


def _prep_body(pd0_ref, pd1_ref, x_ref, dinv_ref, yt_ref):
  deg = 1.0 + pd0_ref[:, 0:1] + pd1_ref[:, 0:1]
  dinv = lax.rsqrt(deg)
  dinv_ref[...] = dinv
  yt_ref[...] = x_ref[...] * dinv


def _layer1_body(p0_ref, p1_ref, yt_ref, dinv_ref, w_ref, b_ref, out_ref):
  agg = (p0_ref[...] + p1_ref[...] + yt_ref[...]) * dinv_ref[...]
  h = jnp.dot(agg, w_ref[...], preferred_element_type=jnp.float32)
  h = jnp.maximum(h + b_ref[...], 0.0)
  out_ref[...] = h * dinv_ref[...]


def _head_body(q0_ref, q1_ref, yt_ref, dinv_ref, wmu_ref, bmu_ref, wlv_ref,
               blv_ref, mu_ref, lv_ref):
  agg = (q0_ref[...] + q1_ref[...] + yt_ref[...]) * dinv_ref[...]
  mu_ref[...] = (
      jnp.dot(agg, wmu_ref[...], preferred_element_type=jnp.float32)
      + bmu_ref[...]
  )
  lv_ref[...] = (
      jnp.dot(agg, wlv_ref[...], preferred_element_type=jnp.float32)
      + blv_ref[...]
  )


def kernel(x, edge_index, W1, b1, Wmu, bmu, Wlv, blv):
  n_nodes, in_dim = x.shape
  hid = W1.shape[1]
  z = Wmu.shape[1]
  n_edges = edge_index.shape[1]

  nw = NC * NS
  chunks_per_w = -(-n_edges // (nw * CHUNK))
  e_pad = nw * chunks_per_w * CHUNK

  src = edge_index[0].astype(jnp.int32)
  dst = edge_index[1].astype(jnp.int32)
  pad = jnp.full((e_pad - n_edges,), n_nodes, dtype=jnp.int32)
  srcp = jnp.concatenate([src, pad]).reshape(nw, chunks_per_w, CHUNK)
  dstp = jnp.concatenate([dst, pad]).reshape(nw, chunks_per_w, CHUNK)

  x_pad = jnp.zeros((N_PAD, in_dim), x.dtype).at[:n_nodes].set(x)
  zeros_d = jnp.zeros((N_PAD, in_dim), jnp.float32)
  zeros_deg = jnp.zeros((N_PAD, DEG_W), jnp.float32)
  ones_deg = jnp.ones((CHUNK, DEG_W), jnp.float32)

  sc_deg = _make_sc_deg(chunks_per_w)
  sc_scatter = _make_sc_scatter(chunks_per_w, in_dim)

  degp = sc_deg(ones_deg, dstp, zeros_deg)
  pd0 = degp[:N_PAD]
  pd1 = degp[N_PAD:]

  bm = 1024
  grid = (N_PAD // bm,)
  row_spec = lambda d: pl.BlockSpec((bm, d), lambda i: (i, 0))
  full_spec = lambda a, b: pl.BlockSpec((a, b), lambda i: (0, 0))

  dinv, yt1 = pl.pallas_call(
      _prep_body,
      grid=grid,
      in_specs=[row_spec(DEG_W), row_spec(DEG_W), row_spec(in_dim)],
      out_specs=[row_spec(1), row_spec(in_dim)],
      out_shape=[
          jax.ShapeDtypeStruct((N_PAD, 1), jnp.float32),
          jax.ShapeDtypeStruct((N_PAD, in_dim), jnp.float32),
      ],
  )(pd0, pd1, x_pad)

  p = sc_scatter(yt1, srcp, dstp, zeros_d)

  yt2 = pl.pallas_call(
      _layer1_body,
      grid=grid,
      in_specs=[
          row_spec(in_dim),
          row_spec(in_dim),
          row_spec(in_dim),
          row_spec(1),
          full_spec(in_dim, hid),
          full_spec(1, hid),
      ],
      out_specs=row_spec(hid),
      out_shape=jax.ShapeDtypeStruct((N_PAD, hid), jnp.float32),
  )(p[:N_PAD], p[N_PAD:], yt1, dinv, W1, b1.reshape(1, hid))

  q = sc_scatter(yt2, srcp, dstp, zeros_d)

  mu, lv = pl.pallas_call(
      _head_body,
      grid=grid,
      in_specs=[
          row_spec(hid),
          row_spec(hid),
          row_spec(hid),
          row_spec(1),
          full_spec(hid, z),
          full_spec(1, z),
          full_spec(hid, z),
          full_spec(1, z),
      ],
      out_specs=[row_spec(z), row_spec(z)],
      out_shape=[
          jax.ShapeDtypeStruct((N_PAD, z), jnp.float32),
          jax.ShapeDtypeStruct((N_PAD, z), jnp.float32),
      ],
  )(q[:N_PAD], q[N_PAD:], yt2, dinv, Wmu, bmu.reshape(1, z), Wlv,
    blv.reshape(1, z))

  return (mu[:n_nodes], lv[:n_nodes])
